# Initial kernel scaffold; baseline (speedup 1.0000x reference)
import jax
import jax.numpy as jnp
from jax.experimental import pallas as pl


def kernel(x, edge_index, edge_type, enc_W, enc_Wg, enc_bg, dec_W):
    raise NotImplementedError("write your pallas kernel here")


# speedup vs baseline: 5.8038x; 5.8038x over previous
import functools

import jax
import jax.numpy as jnp
from jax import lax
from jax.experimental import pallas as pl
from jax.experimental.pallas import tpu as pltpu
from jax.experimental.pallas import tpu_sc as plsc

NC = 2
NS = 16
EB = 128
DH = 64
RELS = 3


def _sc_mesh():
    return plsc.VectorSubcoreMesh(core_axis_name="c", subcore_axis_name="s")


def _scatter_kernel(n_fused, blk, ch):
    stripe = n_fused // NS
    n_chunks = stripe // ch

    def body(h0_hbm, h1_hbm, sidx_hbm, fidx_hbm, zeros_hbm, s0_hbm, s1_hbm,
             idx_s, idx_f, rows_v, acc_sh, sem):
        cid = lax.axis_index("c")
        sid = lax.axis_index("s")

        def run(h_hbm, out_hbm):
            pltpu.sync_copy(
                zeros_hbm, acc_sh.at[pl.ds(sid * stripe, stripe)])
            plsc.subcore_barrier()

            @pl.loop(0, blk)
            def _(j):
                pltpu.sync_copy(sidx_hbm.at[sid, j], idx_s)
                pltpu.sync_copy(fidx_hbm.at[sid, j], idx_f)
                pltpu.async_copy(h_hbm.at[idx_s], rows_v, sem).wait()
                pltpu.sync_copy(rows_v, acc_sh.at[idx_f], add=True)

            plsc.subcore_barrier()
            pltpu.sync_copy(acc_sh.at[pl.ds(sid * stripe, stripe)],
                            out_hbm.at[pl.ds(sid * stripe, stripe)])

        @pl.when(cid == 0)
        def _():
            run(h0_hbm, s0_hbm)

        @pl.when(cid == 1)
        def _():
            run(h1_hbm, s1_hbm)

    return pl.kernel(
        body,
        out_type=(jax.ShapeDtypeStruct((n_fused, DH), jnp.float32),
                  jax.ShapeDtypeStruct((n_fused, DH), jnp.float32)),
        mesh=_sc_mesh(),
        scratch_types=[
            pltpu.VMEM((EB,), jnp.int32),
            pltpu.VMEM((EB,), jnp.int32),
            pltpu.VMEM((EB, DH), jnp.float32),
            pltpu.VMEM_SHARED((n_fused, DH), jnp.float32),
            pltpu.SemaphoreType.DMA,
        ],
        compiler_params=pltpu.CompilerParams(use_tc_tiling_on_sc=False),
    )


def _tc_layer_body(gated, s0_ref, s1_ref, inv_ref, h0_ref, h1_ref,
                   wa_ref, wb_ref, wg0_ref, wg1_ref, bg_ref,
                   o0_ref, o1_ref):
    f32 = jnp.float32
    a = jnp.dot(s0_ref[...] * inv_ref[...], wa_ref[...],
                preferred_element_type=f32)
    a = a + jnp.dot(s1_ref[...] * inv_ref[...], wb_ref[...],
                    preferred_element_type=f32)
    if gated:
        g = jnp.dot(h0_ref[...], wg0_ref[...], preferred_element_type=f32)
        g = g + jnp.dot(h1_ref[...], wg1_ref[...], preferred_element_type=f32)
        g = jax.nn.sigmoid(g + bg_ref[...])
        out = jnp.maximum(g * a, 0.0)
    else:
        out = a - jnp.tanh(a)
    o0_ref[...] = out[:, :DH]
    o1_ref[...] = out[:, DH:]


def _tc_layer(n_pad, gated, bn):
    kdim = RELS * DH
    grid = (n_pad // bn,)
    row_blk = lambda w: pl.BlockSpec((bn, w), lambda i: (i, 0))
    full = lambda a, b: pl.BlockSpec((a, b), lambda i: (0, 0))
    return pl.pallas_call(
        functools.partial(_tc_layer_body, gated),
        grid=grid,
        in_specs=[
            row_blk(kdim), row_blk(kdim), row_blk(kdim),
            row_blk(DH), row_blk(DH),
            full(kdim, 2 * DH), full(kdim, 2 * DH),
            full(DH, 2 * DH), full(DH, 2 * DH), full(1, 2 * DH),
        ],
        out_specs=[row_blk(DH), row_blk(DH)],
        out_shape=(jax.ShapeDtypeStruct((n_pad, DH), jnp.float32),
                   jax.ShapeDtypeStruct((n_pad, DH), jnp.float32)),
    )


def kernel(x, edge_index, edge_type, enc_W, enc_Wg, enc_bg, dec_W):
    n, d = x.shape
    e = edge_index.shape[1]
    assert d == 2 * DH

    n_pad = 10112 if n <= 10112 else ((n + 15) // 16) * 16
    bn = n_pad // 8
    n_fused = n_pad * RELS
    ch = n_fused // NS
    blk = -(-e // (NS * EB))
    e_pad = NS * EB * blk

    src = edge_index[0]
    dst = edge_index[1]
    fused = dst * RELS + edge_type
    pad = e_pad - e
    src_p = jnp.concatenate(
        [src, jnp.zeros((pad,), jnp.int32)]).reshape(NS, blk, EB)
    fused_p = jnp.concatenate(
        [fused, jnp.full((pad,), n * RELS, jnp.int32)]).reshape(NS, blk, EB)

    zeros_ch = jnp.zeros((ch, DH), jnp.float32)
    scatter = _scatter_kernel(n_fused, blk, ch)

    ones_tab = jnp.ones((n_pad, DH), jnp.float32)
    cnt, _ = scatter(ones_tab, ones_tab, src_p, fused_p, zeros_ch)
    inv = 1.0 / jnp.maximum(cnt[:, 0], 1.0)
    inv_e = jnp.repeat(inv.reshape(n_pad, RELS), DH, axis=1)

    x_pad = jnp.zeros((n_pad, d), x.dtype).at[:n].set(x)
    h0, h1 = x_pad[:, :DH], x_pad[:, DH:]

    def agg_inputs(h0, h1, W):
        s0, s1 = scatter(h0, h1, src_p, fused_p, zeros_ch)
        wa = W[:, :DH, :].reshape(RELS * DH, d)
        wb = W[:, DH:, :].reshape(RELS * DH, d)
        return (s0.reshape(n_pad, RELS * DH), s1.reshape(n_pad, RELS * DH),
                inv_e, h0, h1, wa, wb)

    enc = _tc_layer(n_pad, gated=True, bn=bn)
    dec = _tc_layer(n_pad, gated=False, bn=bn)
    zg = jnp.zeros((DH, d), jnp.float32)
    zb = jnp.zeros((1, d), jnp.float32)

    for l in range(enc_W.shape[0]):
        wg = enc_Wg[l]
        h0, h1 = enc(*agg_inputs(h0, h1, enc_W[l]),
                     wg[:DH], wg[DH:], enc_bg[l].reshape(1, d))
    for l in range(dec_W.shape[0]):
        h0, h1 = dec(*agg_inputs(h0, h1, dec_W[l]), zg, zg, zb)

    return jnp.concatenate([h0, h1], axis=1)[:n]


# speedup vs baseline: 6.8447x; 1.1793x over previous
import functools

import jax
import jax.numpy as jnp
from jax import lax
from jax.experimental import pallas as pl
from jax.experimental.pallas import tpu as pltpu
from jax.experimental.pallas import tpu_sc as plsc

NC = 2
NS = 16
EB = 128
DH = 64
RELS = 3


def _sc_mesh():
    return plsc.VectorSubcoreMesh(core_axis_name="c", subcore_axis_name="s")


def _scatter_kernel(n_fused, blk):
    stripe = n_fused // NS
    assert blk % 2 == 0

    def body(h0_hbm, h1_hbm, sf_hbm, zeros_hbm, s0_hbm, s1_hbm, acc_sh):
        cid = lax.axis_index("c")
        sid = lax.axis_index("s")

        def run(h_hbm, out_hbm):
            pltpu.sync_copy(
                zeros_hbm, acc_sh.at[pl.ds(sid * stripe, stripe)])
            plsc.subcore_barrier()

            @plsc.parallel_loop(0, blk, unroll=2)
            def _(j):
                def scoped(idx_v, rows_v, sem):
                    pltpu.sync_copy(sf_hbm.at[sid, j], idx_v)
                    pltpu.async_copy(
                        h_hbm.at[idx_v.at[0]], rows_v, sem).wait()
                    pltpu.sync_copy(
                        rows_v, acc_sh.at[idx_v.at[1]], add=True)
                pl.run_scoped(
                    scoped,
                    pltpu.VMEM((2, EB), jnp.int32),
                    pltpu.VMEM((EB, DH), jnp.float32),
                    pltpu.SemaphoreType.DMA)

            plsc.subcore_barrier()
            pltpu.sync_copy(acc_sh.at[pl.ds(sid * stripe, stripe)],
                            out_hbm.at[pl.ds(sid * stripe, stripe)])

        @pl.when(cid == 0)
        def _():
            run(h0_hbm, s0_hbm)

        @pl.when(cid == 1)
        def _():
            run(h1_hbm, s1_hbm)

    return pl.kernel(
        body,
        out_type=(jax.ShapeDtypeStruct((n_fused, DH), jnp.float32),
                  jax.ShapeDtypeStruct((n_fused, DH), jnp.float32)),
        mesh=_sc_mesh(),
        scratch_types=[
            pltpu.VMEM_SHARED((n_fused, DH), jnp.float32),
        ],
        compiler_params=pltpu.CompilerParams(use_tc_tiling_on_sc=False),
    )


def _counts_kernel(n_fused, blk):
    stripe = n_fused // NS

    def body(sf_hbm, ones_hbm, zeros_hbm, cnt_hbm, idx_v, ones_v, acc_sh, sem):
        cid = lax.axis_index("c")
        sid = lax.axis_index("s")

        @pl.when(cid == 0)
        def _():
            pltpu.sync_copy(
                zeros_hbm, acc_sh.at[pl.ds(sid * stripe, stripe)])
            pltpu.sync_copy(ones_hbm, ones_v)
            plsc.subcore_barrier()

            @pl.loop(0, blk)
            def _(j):
                pltpu.sync_copy(sf_hbm.at[sid, j], idx_v)
                pltpu.sync_copy(ones_v, acc_sh.at[idx_v.at[1]], add=True)

            plsc.subcore_barrier()
            pltpu.sync_copy(acc_sh.at[pl.ds(sid * stripe, stripe)],
                            cnt_hbm.at[pl.ds(sid * stripe, stripe)])

    return pl.kernel(
        body,
        out_type=jax.ShapeDtypeStruct((n_fused, 16), jnp.float32),
        mesh=_sc_mesh(),
        scratch_types=[
            pltpu.VMEM((2, EB), jnp.int32),
            pltpu.VMEM((EB, 16), jnp.float32),
            pltpu.VMEM_SHARED((n_fused, 16), jnp.float32),
            pltpu.SemaphoreType.DMA,
        ],
        compiler_params=pltpu.CompilerParams(use_tc_tiling_on_sc=False),
    )


def _tc_layer_body(gated, s0_ref, s1_ref, inv_ref, h0_ref, h1_ref,
                   wa_ref, wb_ref, wg0_ref, wg1_ref, bg_ref,
                   o0_ref, o1_ref):
    f32 = jnp.float32
    a = jnp.dot(s0_ref[...] * inv_ref[...], wa_ref[...],
                preferred_element_type=f32)
    a = a + jnp.dot(s1_ref[...] * inv_ref[...], wb_ref[...],
                    preferred_element_type=f32)
    if gated:
        g = jnp.dot(h0_ref[...], wg0_ref[...], preferred_element_type=f32)
        g = g + jnp.dot(h1_ref[...], wg1_ref[...], preferred_element_type=f32)
        g = jax.nn.sigmoid(g + bg_ref[...])
        out = jnp.maximum(g * a, 0.0)
    else:
        out = a - jnp.tanh(a)
    o0_ref[...] = out[:, :DH]
    o1_ref[...] = out[:, DH:]


def _tc_layer(n_pad, gated, bn):
    kdim = RELS * DH
    grid = (n_pad // bn,)
    row_blk = lambda w: pl.BlockSpec((bn, w), lambda i: (i, 0))
    full = lambda a, b: pl.BlockSpec((a, b), lambda i: (0, 0))
    return pl.pallas_call(
        functools.partial(_tc_layer_body, gated),
        grid=grid,
        in_specs=[
            row_blk(kdim), row_blk(kdim), row_blk(kdim),
            row_blk(DH), row_blk(DH),
            full(kdim, 2 * DH), full(kdim, 2 * DH),
            full(DH, 2 * DH), full(DH, 2 * DH), full(1, 2 * DH),
        ],
        out_specs=[row_blk(DH), row_blk(DH)],
        out_shape=(jax.ShapeDtypeStruct((n_pad, DH), jnp.float32),
                   jax.ShapeDtypeStruct((n_pad, DH), jnp.float32)),
    )


def kernel(x, edge_index, edge_type, enc_W, enc_Wg, enc_bg, dec_W):
    n, d = x.shape
    e = edge_index.shape[1]
    assert d == 2 * DH

    n_pad = 10112 if n <= 10112 else ((n + 15) // 16) * 16
    bn = n_pad // 8
    n_fused = n_pad * RELS
    stripe = n_fused // NS
    blk = 2 * (-(-e // (NS * EB * 2)))
    e_pad = NS * EB * blk

    src = edge_index[0]
    dst = edge_index[1]
    fused = dst * RELS + edge_type
    pad = e_pad - e
    src_p = jnp.concatenate(
        [src, jnp.zeros((pad,), jnp.int32)]).reshape(NS, blk, 1, EB)
    fused_p = jnp.concatenate(
        [fused, jnp.full((pad,), n * RELS, jnp.int32)]).reshape(NS, blk, 1, EB)
    sf_p = jnp.concatenate([src_p, fused_p], axis=2)

    zeros_st = jnp.zeros((stripe, DH), jnp.float32)
    zeros16 = jnp.zeros((stripe, 16), jnp.float32)
    ones16 = jnp.ones((EB, 16), jnp.float32)
    scatter = _scatter_kernel(n_fused, blk)

    cnt = _counts_kernel(n_fused, blk)(sf_p, ones16, zeros16)
    inv = 1.0 / jnp.maximum(cnt[:, 0], 1.0)
    inv_e = jnp.repeat(inv.reshape(n_pad, RELS), DH, axis=1)

    x_pad = jnp.zeros((n_pad, d), x.dtype).at[:n].set(x)
    h0, h1 = x_pad[:, :DH], x_pad[:, DH:]

    def agg_inputs(h0, h1, W):
        s0, s1 = scatter(h0, h1, sf_p, zeros_st)
        wa = W[:, :DH, :].reshape(RELS * DH, d)
        wb = W[:, DH:, :].reshape(RELS * DH, d)
        return (s0.reshape(n_pad, RELS * DH), s1.reshape(n_pad, RELS * DH),
                inv_e, h0, h1, wa, wb)

    enc = _tc_layer(n_pad, gated=True, bn=bn)
    dec = _tc_layer(n_pad, gated=False, bn=bn)
    zg = jnp.zeros((DH, d), jnp.float32)
    zb = jnp.zeros((1, d), jnp.float32)

    for l in range(enc_W.shape[0]):
        wg = enc_Wg[l]
        h0, h1 = enc(*agg_inputs(h0, h1, enc_W[l]),
                     wg[:DH], wg[DH:], enc_bg[l].reshape(1, d))
    for l in range(dec_W.shape[0]):
        h0, h1 = dec(*agg_inputs(h0, h1, dec_W[l]), zg, zg, zb)

    return jnp.concatenate([h0, h1], axis=1)[:n]
